# Initial kernel scaffold; baseline (speedup 1.0000x reference)
import functools

import jax
import jax.numpy as jnp
from jax import lax
from jax.experimental import pallas as pl
from jax.experimental.pallas import tpu as pltpu
from jax.experimental.pallas import tpu_sc as plsc

B, L, V, S, D = 1024, 200, 100000, 2, 128

_info = plsc.get_sparse_core_info()
NC, NS, LN = _info.num_cores, _info.num_subcores, _info.num_lanes
NW = NC * NS
ROWS = B * L
RPW = ROWS // NW
G = 128
NG = RPW // G

_mesh = plsc.VectorSubcoreMesh(core_axis_name="c", subcore_axis_name="s")


@functools.partial(
    pl.kernel,
    mesh=_mesh,
    out_type=jax.ShapeDtypeStruct((ROWS, D), jnp.float32),
    scratch_types=[
        pltpu.VMEM((RPW,), jnp.int32),
        pltpu.VMEM((RPW,), jnp.int32),
        pltpu.VMEM((G, D), jnp.float32),
        pltpu.VMEM((G, D), jnp.float32),
        pltpu.SemaphoreType.DMA,
        pltpu.SemaphoreType.DMA,
    ],
)
def _emb_kernel(tok_hbm, posseg_hbm, tokens_hbm, segments_hbm, out_hbm,
                tokidx_v, psidx_v, tokbuf, psbuf, sem_t, sem_p):
    wid = lax.axis_index("s") * NC + lax.axis_index("c")
    base = wid * RPW

    pltpu.sync_copy(tokens_hbm.at[wid], tokidx_v)
    pltpu.sync_copy(segments_hbm.at[wid], psidx_v)

    def idx_body(k, _):
        s16 = psidx_v[pl.ds(k * LN, LN)]
        p = lax.iota(jnp.int32, (LN,)) + (base + k * LN)
        l = lax.rem(p, L)
        psidx_v[pl.ds(k * LN, LN)] = 2 * l + s16
        return 0

    lax.fori_loop(0, RPW // LN, idx_body, 0)

    def group_body(g, _):
        cp_t = pltpu.async_copy(
            tok_hbm.at[tokidx_v.at[pl.ds(g * G, G)]], tokbuf, sem_t)
        cp_p = pltpu.async_copy(
            posseg_hbm.at[psidx_v.at[pl.ds(g * G, G)]], psbuf, sem_p)
        cp_t.wait()
        cp_p.wait()

        def add_body(r, _):
            for c in range(D // LN):
                tokbuf[r, pl.ds(c * LN, LN)] = (
                    tokbuf[r, pl.ds(c * LN, LN)] + psbuf[r, pl.ds(c * LN, LN)])
            return 0

        lax.fori_loop(0, G, add_body, 0)
        pltpu.sync_copy(tokbuf, out_hbm.at[pl.ds(base + g * G, G)])
        return 0

    lax.fori_loop(0, NG, group_body, 0)


def kernel(tokens, segments, tok_table, pos_table, seg_table):
    tok_z = tok_table.at[0].set(0.0)
    seg_z = seg_table.at[0].set(0.0)
    posseg = (pos_table[:, None, :] + seg_z[None, :, :]).reshape(L * S, D)
    out = _emb_kernel(
        tok_z,
        posseg,
        tokens.reshape(NW, RPW).astype(jnp.int32),
        segments.reshape(NW, RPW).astype(jnp.int32),
    )
    return out.reshape(B, L, D)


# speedup vs baseline: 5.4557x; 5.4557x over previous
import functools

import jax
import jax.numpy as jnp
from jax import lax
from jax.experimental import pallas as pl
from jax.experimental.pallas import tpu as pltpu
from jax.experimental.pallas import tpu_sc as plsc

B, L, V, S, D = 1024, 200, 100000, 2, 128

_info = plsc.get_sparse_core_info()
NC, NS, LN = _info.num_cores, _info.num_subcores, _info.num_lanes
NW = NC * NS
ROWS = B * L
RPW = ROWS // NW
G = 128
NG = RPW // G

_mesh = plsc.VectorSubcoreMesh(core_axis_name="c", subcore_axis_name="s")


@functools.partial(
    pl.kernel,
    mesh=_mesh,
    out_type=jax.ShapeDtypeStruct((ROWS, D), jnp.float32),
    scratch_types=[
        pltpu.VMEM((RPW,), jnp.int32),
        pltpu.VMEM((RPW,), jnp.int32),
        pltpu.VMEM((G, D), jnp.float32),
        pltpu.VMEM((G, D), jnp.float32),
        pltpu.SemaphoreType.DMA,
        pltpu.SemaphoreType.DMA,
    ],
)
def _emb_kernel(tok_hbm, posseg_hbm, tokens_hbm, segments_hbm, out_hbm,
                tokidx_v, psidx_v, tokbuf, psbuf, sem_t, sem_p):
    wid = lax.axis_index("s") * NC + lax.axis_index("c")
    base = wid * RPW

    pltpu.sync_copy(tokens_hbm.at[wid], tokidx_v)
    pltpu.sync_copy(segments_hbm.at[wid], psidx_v)

    def idx_body(k, _):
        s16 = psidx_v[pl.ds(k * LN, LN)]
        p = lax.iota(jnp.int32, LN) + (base + k * LN)
        l = lax.rem(p, L)
        psidx_v[pl.ds(k * LN, LN)] = 2 * l + s16
        return 0

    lax.fori_loop(0, RPW // LN, idx_body, 0)

    def group_body(g, _):
        cp_t = pltpu.async_copy(
            tok_hbm.at[tokidx_v.at[pl.ds(g * G, G)]], tokbuf, sem_t)
        cp_p = pltpu.async_copy(
            posseg_hbm.at[psidx_v.at[pl.ds(g * G, G)]], psbuf, sem_p)
        cp_t.wait()
        cp_p.wait()

        def add_body(r, _):
            for c in range(D // LN):
                tokbuf[r, pl.ds(c * LN, LN)] = (
                    tokbuf[r, pl.ds(c * LN, LN)] + psbuf[r, pl.ds(c * LN, LN)])
            return 0

        lax.fori_loop(0, G, add_body, 0)
        pltpu.sync_copy(tokbuf, out_hbm.at[pl.ds(base + g * G, G)])
        return 0

    lax.fori_loop(0, NG, group_body, 0)


def kernel(tokens, segments, tok_table, pos_table, seg_table):
    tok_z = tok_table.at[0].set(0.0)
    seg_z = seg_table.at[0].set(0.0)
    posseg = (pos_table[:, None, :] + seg_z[None, :, :]).reshape(L * S, D)
    out = _emb_kernel(
        tok_z,
        posseg,
        tokens.reshape(NW, RPW).astype(jnp.int32),
        segments.reshape(NW, RPW).astype(jnp.int32),
    )
    return out.reshape(B, L, D)


# speedup vs baseline: 5.6079x; 1.0279x over previous
import functools

import jax
import jax.numpy as jnp
from jax import lax
from jax.experimental import pallas as pl
from jax.experimental.pallas import tpu as pltpu
from jax.experimental.pallas import tpu_sc as plsc

B, L, V, S, D = 1024, 200, 100000, 2, 128

_info = plsc.get_sparse_core_info()
NC, NS, LN = _info.num_cores, _info.num_subcores, _info.num_lanes
NW = NC * NS
ROWS = B * L
RPW = ROWS // NW
G = 64
NG = RPW // G
KV = G // LN

_mesh = plsc.VectorSubcoreMesh(core_axis_name="c", subcore_axis_name="s")

_DNUMS = lax.GatherDimensionNumbers(
    offset_dims=(), collapsed_slice_dims=(0,), start_index_map=(0,))


def _bcast_lane(vec, lane):
    idx = jnp.full((LN, 1), lane, jnp.int32)
    return lax.gather(vec, idx, dimension_numbers=_DNUMS, slice_sizes=(1,),
                      mode=lax.GatherScatterMode.PROMISE_IN_BOUNDS)


@functools.partial(
    pl.kernel,
    mesh=_mesh,
    out_type=jax.ShapeDtypeStruct((ROWS, D), jnp.float32),
    scratch_types=[
        pltpu.VMEM((NG, G), jnp.int32),
        pltpu.VMEM((NG, G), jnp.int32),
        pltpu.VMEM((NG, G), jnp.float32),
        pltpu.VMEM((G, D), jnp.float32),
        pltpu.VMEM((G, D), jnp.float32),
        pltpu.VMEM((G, D), jnp.float32),
        pltpu.VMEM((G, D), jnp.float32),
        pltpu.VMEM((G, D), jnp.float32),
        pltpu.VMEM((G, D), jnp.float32),
        pltpu.SemaphoreType.DMA,
        pltpu.SemaphoreType.DMA,
        pltpu.SemaphoreType.DMA,
        pltpu.SemaphoreType.DMA,
        pltpu.SemaphoreType.DMA,
        pltpu.SemaphoreType.DMA,
    ],
)
def _emb_kernel(tok_hbm, posseg_hbm, tokens_hbm, segments_hbm, out_hbm,
                tokidx, psidx, maskf, tb0, tb1, pb0, pb1, ob0, ob1,
                st0, st1, sp0, sp1, so0, so1):
    wid = lax.axis_index("s") * NC + lax.axis_index("c")
    base = wid * RPW
    bufs = ((tb0, pb0, ob0, st0, sp0, so0), (tb1, pb1, ob1, st1, sp1, so1))

    pltpu.sync_copy(tokens_hbm.at[wid], tokidx)
    pltpu.sync_copy(segments_hbm.at[wid], psidx)

    iota = lax.iota(jnp.int32, LN)

    def prep_body(gg, _):
        for kk in range(KV):
            off = kk * LN
            t16 = tokidx[gg, pl.ds(off, LN)]
            s16 = psidx[gg, pl.ds(off, LN)]
            p = iota + (base + off) + gg * G
            l = lax.rem(p, L)
            psidx[gg, pl.ds(off, LN)] = 2 * l + s16
            maskf[gg, pl.ds(off, LN)] = jnp.where(t16 == 0, 0.0, 1.0)
        return 0

    lax.fori_loop(0, NG, prep_body, 0)

    def issue_gathers(g, b):
        tb, pb, _, st, sp, _ = bufs[b]
        pltpu.async_copy(tok_hbm.at[tokidx.at[g]], tb, st)
        pltpu.async_copy(posseg_hbm.at[psidx.at[g]], pb, sp)

    issue_gathers(0, 0)
    issue_gathers(1, 1)

    def pair_body(i, _):
        for b in range(2):
            g = i * 2 + b
            tb, pb, ob, st, sp, so = bufs[b]
            pltpu.make_async_copy(tok_hbm.at[tokidx.at[g]], tb, st).wait()
            pltpu.make_async_copy(posseg_hbm.at[psidx.at[g]], pb, sp).wait()

            @pl.when(g >= 2)
            def _drain():
                pltpu.make_async_copy(
                    ob, out_hbm.at[pl.ds(base + (g - 2) * G, G)], so).wait()

            def add_block(jj, _):
                mf16 = maskf[g, pl.ds(jj * LN, LN)]
                for rr in range(LN):
                    r = jj * LN + rr
                    mrow = _bcast_lane(mf16, rr)
                    for c in range(D // LN):
                        sl = pl.ds(c * LN, LN)
                        ob[r, sl] = tb[r, sl] * mrow + pb[r, sl]
                return 0

            lax.fori_loop(0, KV, add_block, 0)
            pltpu.async_copy(ob, out_hbm.at[pl.ds(base + g * G, G)], so)

            @pl.when(g + 2 < NG)
            def _next():
                issue_gathers(g + 2, b)
        return 0

    lax.fori_loop(0, NG // 2, pair_body, 0)

    for b in range(2):
        g_last = NG - 2 + b
        _, _, ob, _, _, so = bufs[b]
        pltpu.make_async_copy(
            ob, out_hbm.at[pl.ds(base + g_last * G, G)], so).wait()


def kernel(tokens, segments, tok_table, pos_table, seg_table):
    seg_z = seg_table.at[0].set(0.0)
    posseg = (pos_table[:, None, :] + seg_z[None, :, :]).reshape(L * S, D)
    out = _emb_kernel(
        tok_table,
        posseg,
        tokens.reshape(NW, NG, G).astype(jnp.int32),
        segments.reshape(NW, NG, G).astype(jnp.int32),
    )
    return out.reshape(B, L, D)


# speedup vs baseline: 5.7388x; 1.0233x over previous
import functools

import jax
import jax.numpy as jnp
from jax import lax
from jax.experimental import pallas as pl
from jax.experimental.pallas import tpu as pltpu
from jax.experimental.pallas import tpu_sc as plsc

B, L, V, S, D = 1024, 200, 100000, 2, 128

_info = plsc.get_sparse_core_info()
NC, NS, LN = _info.num_cores, _info.num_subcores, _info.num_lanes
NW = NC * NS
ROWS = B * L
RPW = ROWS // NW
G = 128
NG = RPW // G
KV = G // LN
RU = 4

_mesh = plsc.VectorSubcoreMesh(core_axis_name="c", subcore_axis_name="s")


@functools.partial(
    pl.kernel,
    mesh=_mesh,
    out_type=jax.ShapeDtypeStruct((ROWS, D), jnp.float32),
    scratch_types=[
        pltpu.VMEM((NG, G), jnp.int32),
        pltpu.VMEM((NG, G), jnp.int32),
        pltpu.VMEM((G, D), jnp.float32),
        pltpu.VMEM((G, D), jnp.float32),
        pltpu.VMEM((G, D), jnp.float32),
        pltpu.VMEM((G, D), jnp.float32),
        pltpu.VMEM((G, D), jnp.float32),
        pltpu.VMEM((G, D), jnp.float32),
        pltpu.SemaphoreType.DMA,
        pltpu.SemaphoreType.DMA,
        pltpu.SemaphoreType.DMA,
        pltpu.SemaphoreType.DMA,
        pltpu.SemaphoreType.DMA,
        pltpu.SemaphoreType.DMA,
    ],
)
def _emb_kernel(tok_hbm, psx_hbm, tokens_hbm, segments_hbm, out_hbm,
                tokidx, psidx, tb0, tb1, pb0, pb1, ob0, ob1,
                st0, st1, sp0, sp1, so0, so1):
    wid = lax.axis_index("s") * NC + lax.axis_index("c")
    base = wid * RPW
    bufs = ((tb0, pb0, ob0, st0, sp0, so0), (tb1, pb1, ob1, st1, sp1, so1))

    pltpu.sync_copy(tokens_hbm.at[wid], tokidx)
    pltpu.sync_copy(segments_hbm.at[wid], psidx)

    iota = lax.iota(jnp.int32, LN)

    def prep_body(gg, _):
        for kk in range(KV):
            off = kk * LN
            t16 = tokidx[gg, pl.ds(off, LN)]
            s16 = psidx[gg, pl.ds(off, LN)]
            p = iota + (base + off) + gg * G
            l = lax.rem(p, L)
            psidx[gg, pl.ds(off, LN)] = (
                2 * l + s16 + jnp.where(t16 == 0, S * L, 0))
        return 0

    lax.fori_loop(0, NG, prep_body, 0)

    def issue_gathers(g, b):
        tb, pb, _, st, sp, _ = bufs[b]
        pltpu.async_copy(tok_hbm.at[tokidx.at[g]], tb, st)
        pltpu.async_copy(psx_hbm.at[psidx.at[g]], pb, sp)

    issue_gathers(0, 0)
    issue_gathers(1, 1)

    def pair_body(i, _):
        for b in range(2):
            g = i * 2 + b
            tb, pb, ob, st, sp, so = bufs[b]
            pltpu.make_async_copy(tok_hbm.at[tokidx.at[g]], tb, st).wait()
            pltpu.make_async_copy(psx_hbm.at[psidx.at[g]], pb, sp).wait()

            @pl.when(g >= 2)
            def _drain():
                pltpu.make_async_copy(
                    ob, out_hbm.at[pl.ds(base + (g - 2) * G, G)], so).wait()

            def add_block(jj, _):
                for rr in range(RU):
                    r = jj * RU + rr
                    for c in range(D // LN):
                        sl = pl.ds(c * LN, LN)
                        ob[r, sl] = tb[r, sl] + pb[r, sl]
                return 0

            lax.fori_loop(0, G // RU, add_block, 0)
            pltpu.async_copy(ob, out_hbm.at[pl.ds(base + g * G, G)], so)

            @pl.when(g + 2 < NG)
            def _next():
                issue_gathers(g + 2, b)
        return 0

    lax.fori_loop(0, NG // 2, pair_body, 0)

    for b in range(2):
        g_last = NG - 2 + b
        _, _, ob, _, _, so = bufs[b]
        pltpu.make_async_copy(
            ob, out_hbm.at[pl.ds(base + g_last * G, G)], so).wait()


def kernel(tokens, segments, tok_table, pos_table, seg_table):
    seg_z = seg_table.at[0].set(0.0)
    posseg = (pos_table[:, None, :] + seg_z[None, :, :]).reshape(L * S, D)
    psx = jnp.concatenate([posseg, posseg - tok_table[0][None, :]], axis=0)
    out = _emb_kernel(
        tok_table,
        psx,
        tokens.reshape(NW, NG, G).astype(jnp.int32),
        segments.reshape(NW, NG, G).astype(jnp.int32),
    )
    return out.reshape(B, L, D)
